# baseline (speedup 1.0000x reference)
import functools

import jax
import jax.numpy as jnp
from jax import lax
from jax.experimental import pallas as pl
from jax.experimental.pallas import tpu as pltpu
from jax.experimental.pallas import tpu_sc as plsc

f32 = jnp.float32
i32 = jnp.int32

N = 10000
NPAD = 10240
DUMMY = 10000
MBLK = 256
NC, NS, NW = 2, 16, 32
EB = 256
E1P = 106496
E4P = 106496
EGP = 425984
RPS = NPAD // NS


_SC_PARAMS = pltpu.CompilerParams(needs_layout_passes=False)


def _mesh():
    return plsc.VectorSubcoreMesh(core_axis_name="c", subcore_axis_name="s")


def _row_scatter(table, src, dst, w, nchunk, fc):
    ep = src.shape[0]
    weighted = w is not None
    esplit = NW if nchunk == 1 else NS
    et = ep // esplit
    nb = et // EB

    if nchunk == 1:
        out_t = jax.ShapeDtypeStruct((NC, NPAD, fc), f32)
    else:
        out_t = jax.ShapeDtypeStruct((NPAD, nchunk * fc), f32)

    scratch = [
        pltpu.VMEM((EB,), i32),
        pltpu.VMEM((EB,), i32),
        pltpu.VMEM((EB, fc), f32),
        pltpu.VMEM_SHARED((NPAD, fc), f32),
        pltpu.SemaphoreType.DMA,
    ]
    if weighted:
        scratch.append(pltpu.VMEM((EB,), f32))

    def body(table_h, src_h, dst_h, *rest):
        if weighted:
            (w_h, out_h, sidx, didx, rows, acc, sem, wv) = rest
        else:
            (out_h, sidx, didx, rows, acc, sem) = rest
        c = lax.axis_index("c")
        s = lax.axis_index("s")
        zero16 = jnp.zeros((16,), f32)

        tid = s * NC + c if esplit == NW else s

        for q in range(max(nchunk // NC, 1)):
            ck = c + NC * q if nchunk > 1 else 0
            f0 = ck * fc

            def zr_body(r, _):
                for qq in range(fc // 16):
                    rows[r, pl.ds(qq * 16, 16)] = zero16
                return 0

            lax.fori_loop(0, EB, zr_body, 0, unroll=False)
            off = 0
            while off < RPS:
                n_ = min(EB, RPS - off)
                pltpu.sync_copy(rows.at[pl.ds(0, n_)],
                                acc.at[pl.ds(s * RPS + off, n_)])
                off += n_
            plsc.subcore_barrier()

            def e_body(b, _):
                e0 = tid * et + b * EB
                pltpu.sync_copy(src_h.at[pl.ds(e0, EB)], sidx)
                pltpu.sync_copy(dst_h.at[pl.ds(e0, EB)], didx)
                if nchunk > 1:
                    pltpu.async_copy(
                        table_h.at[sidx, pl.ds(f0, fc)], rows, sem
                    ).wait()
                else:
                    pltpu.async_copy(table_h.at[sidx], rows, sem).wait()
                if weighted:
                    pltpu.sync_copy(w_h.at[pl.ds(e0, EB)], wv)

                    def m_body(j, _):
                        wvec = wv[pl.ds(j * 16, 16)]
                        for l in range(16):
                            ws = wvec[l]
                            i = j * 16 + l
                            for qq in range(fc // 16):
                                sl = pl.ds(qq * 16, 16)
                                rows[i, sl] = rows[i, sl] * ws
                        return 0

                    lax.fori_loop(0, EB // 16, m_body, 0, unroll=False)
                pltpu.sync_copy(rows, acc.at[didx], add=True)
                return 0

            lax.fori_loop(0, nb, e_body, 0, unroll=False)
            plsc.subcore_barrier()

            r0_ = s * RPS
            if nchunk == 1:
                pltpu.sync_copy(acc.at[pl.ds(r0_, RPS)],
                                out_h.at[c, pl.ds(r0_, RPS)])
            else:
                pltpu.sync_copy(acc.at[pl.ds(r0_, RPS)],
                                out_h.at[pl.ds(r0_, RPS), pl.ds(f0, fc)])
            plsc.subcore_barrier()

    run = pl.kernel(body, out_type=out_t, mesh=_mesh(),
                    scratch_types=scratch, compiler_params=_SC_PARAMS)
    if weighted:
        return run(table, src, dst, w)
    return run(table, src, dst)


def _gat_edge(a_s, a_d, src, dst):
    ep = src.shape[0]
    et = ep // NW
    nb = et // EB

    out_t = (jax.ShapeDtypeStruct((ep,), f32),
             jax.ShapeDtypeStruct((NW, NPAD), f32))
    scratch = [
        pltpu.VMEM((NPAD,), f32),
        pltpu.VMEM((NPAD,), f32),
        pltpu.VMEM((NPAD,), f32),
        pltpu.VMEM((EB,), i32),
        pltpu.VMEM((EB,), i32),
        pltpu.VMEM((EB,), f32),
    ]

    def body(as_h, ad_h, src_h, dst_h, ex_h, den_h,
             asv, adv, acc, sidx, didx, exb):
        c = lax.axis_index("c")
        s = lax.axis_index("s")
        wid = s * NC + c
        pltpu.sync_copy(as_h, asv)
        pltpu.sync_copy(ad_h, adv)
        zero16 = jnp.zeros((16,), f32)

        def z_body(i, _):
            acc[pl.ds(i * 16, 16)] = zero16
            return 0

        lax.fori_loop(0, NPAD // 16, z_body, 0, unroll=False)

        def mx_body(i, cur):
            va = asv[pl.ds(i * 16, 16)]
            vb = adv[pl.ds(i * 16, 16)]
            return (jnp.maximum(cur[0], va), jnp.maximum(cur[1], vb))

        ma, mb = lax.fori_loop(0, NPAD // 16, mx_body,
                               (jnp.zeros((16,), f32), jnp.zeros((16,), f32)),
                               unroll=False)
        m = jnp.max(ma) + jnp.max(mb)

        def e_body(b, _):
            e0 = wid * et + b * EB
            pltpu.sync_copy(src_h.at[pl.ds(e0, EB)], sidx)
            pltpu.sync_copy(dst_h.at[pl.ds(e0, EB)], didx)
            for j in range(EB // 16):
                sl = pl.ds(j * 16, 16)
                si = sidx[sl]
                di = didx[sl]
                av = plsc.load_gather(asv, [si])
                bv = plsc.load_gather(adv, [di])
                t = av + bv
                al = jnp.maximum(t, 0.2 * t)
                e_ = jnp.exp(al - m)
                exb[sl] = e_
                plsc.addupdate_scatter(acc, [di], e_)
            pltpu.sync_copy(exb, ex_h.at[pl.ds(e0, EB)])
            return 0

        lax.fori_loop(0, nb, e_body, 0, unroll=False)
        pltpu.sync_copy(acc, den_h.at[wid])

    run = pl.kernel(body, out_type=out_t, mesh=_mesh(),
                    scratch_types=scratch, compiler_params=_SC_PARAMS)
    return run(a_s, a_d, src, dst)


def _seg_count(dst):
    ep = dst.shape[0]
    et = ep // NW
    nb = et // EB

    out_t = jax.ShapeDtypeStruct((NW, NPAD), f32)
    scratch = [
        pltpu.VMEM((NPAD,), f32),
        pltpu.VMEM((EB,), i32),
    ]

    def body(dst_h, out_h, acc, didx):
        c = lax.axis_index("c")
        s = lax.axis_index("s")
        wid = s * NC + c
        zero16 = jnp.zeros((16,), f32)
        ones16 = jnp.ones((16,), f32)

        def z_body(i, _):
            acc[pl.ds(i * 16, 16)] = zero16
            return 0

        lax.fori_loop(0, NPAD // 16, z_body, 0, unroll=False)

        def e_body(b, _):
            e0 = wid * et + b * EB
            pltpu.sync_copy(dst_h.at[pl.ds(e0, EB)], didx)
            for j in range(EB // 16):
                di = didx[pl.ds(j * 16, 16)]
                plsc.addupdate_scatter(acc, [di], ones16)
            return 0

        lax.fori_loop(0, nb, e_body, 0, unroll=False)
        pltpu.sync_copy(acc, out_h.at[wid])

    run = pl.kernel(body, out_type=out_t, mesh=_mesh(),
                    scratch_types=scratch, compiler_params=_SC_PARAMS)
    return run(dst)


def _tc1_body(x_ref, w_ref, o_ref):
    o_ref[...] = jnp.dot(x_ref[...], w_ref[...], preferred_element_type=f32)


def _tc1(x, w):
    return pl.pallas_call(
        _tc1_body,
        grid=(2, 40),
        in_specs=[
            pl.BlockSpec((MBLK, 2000), lambda n, m: (m, 0)),
            pl.BlockSpec((2000, 1024), lambda n, m: (0, n)),
        ],
        out_specs=pl.BlockSpec((MBLK, 1024), lambda n, m: (m, n)),
        out_shape=jax.ShapeDtypeStruct((NPAD, 2048), f32),
    )(x, w)


def _tc2_body(s1, cnt, hr, bl1, wlin1, blin1, w2, as2, ad2,
              h2o, aso, ado):
    c = jnp.clip(jnp.sum(cnt[...], axis=0), 1.0, None)
    h1 = jnp.maximum(s1[...] / c[:, None] + bl1[...] + hr[...], 0.0)
    h2 = jnp.maximum(
        jnp.dot(h1, wlin1[...], preferred_element_type=f32) + blin1[...], 0.0)
    hh = jnp.dot(h2, w2[...], preferred_element_type=f32)
    i = pl.program_id(0)
    rows = i * MBLK + lax.broadcasted_iota(i32, (MBLK, 1), 0)
    mask = rows < N
    hh = jnp.where(mask, hh, 0.0)
    h2o[...] = hh
    aso[...] = jnp.dot(hh, as2[...], preferred_element_type=f32)
    ado[...] = jnp.dot(hh, ad2[...], preferred_element_type=f32)


def _tc2(s1, cnt, hr, bl1, wlin1, blin1, w2, as2, ad2):
    return pl.pallas_call(
        _tc2_body,
        grid=(40,),
        in_specs=[
            pl.BlockSpec((MBLK, 1024), lambda i: (i, 0)),
            pl.BlockSpec((NW, MBLK), lambda i: (0, i)),
            pl.BlockSpec((MBLK, 1024), lambda i: (i, 0)),
            pl.BlockSpec((1, 1024), lambda i: (0, 0)),
            pl.BlockSpec((1024, 512), lambda i: (0, 0)),
            pl.BlockSpec((1, 512), lambda i: (0, 0)),
            pl.BlockSpec((512, 256), lambda i: (0, 0)),
            pl.BlockSpec((256, 1), lambda i: (0, 0)),
            pl.BlockSpec((256, 1), lambda i: (0, 0)),
        ],
        out_specs=[
            pl.BlockSpec((MBLK, 256), lambda i: (i, 0)),
            pl.BlockSpec((MBLK, 1), lambda i: (i, 0)),
            pl.BlockSpec((MBLK, 1), lambda i: (i, 0)),
        ],
        out_shape=[
            jax.ShapeDtypeStruct((NPAD, 256), f32),
            jax.ShapeDtypeStruct((NPAD, 1), f32),
            jax.ShapeDtypeStruct((NPAD, 1), f32),
        ],
    )(s1, cnt, hr, bl1.reshape(1, -1), wlin1, blin1.reshape(1, -1), w2,
      as2.reshape(-1, 1), ad2.reshape(-1, 1))


def _tc3_body(o2, den, b2, w3, as3, ad3, h3o, aso, ado):
    d = jnp.clip(jnp.sum(den[...], axis=0), 1e-16, None)
    g2 = jnp.maximum(o2[...] / d[:, None] + b2[...], 0.0)
    h3 = jnp.dot(g2, w3[...], preferred_element_type=f32)
    i = pl.program_id(0)
    rows = i * MBLK + lax.broadcasted_iota(i32, (MBLK, 1), 0)
    mask = rows < N
    h3 = jnp.where(mask, h3, 0.0)
    h3o[...] = h3
    aso[...] = jnp.dot(h3, as3[...], preferred_element_type=f32)
    ado[...] = jnp.dot(h3, ad3[...], preferred_element_type=f32)


def _tc3(o2, den2, b2, w3, as3, ad3):
    return pl.pallas_call(
        _tc3_body,
        grid=(40,),
        in_specs=[
            pl.BlockSpec((MBLK, 256), lambda i: (i, 0)),
            pl.BlockSpec((NW, MBLK), lambda i: (0, i)),
            pl.BlockSpec((1, 256), lambda i: (0, 0)),
            pl.BlockSpec((256, 128), lambda i: (0, 0)),
            pl.BlockSpec((128, 1), lambda i: (0, 0)),
            pl.BlockSpec((128, 1), lambda i: (0, 0)),
        ],
        out_specs=[
            pl.BlockSpec((MBLK, 128), lambda i: (i, 0)),
            pl.BlockSpec((MBLK, 1), lambda i: (i, 0)),
            pl.BlockSpec((MBLK, 1), lambda i: (i, 0)),
        ],
        out_shape=[
            jax.ShapeDtypeStruct((NPAD, 128), f32),
            jax.ShapeDtypeStruct((NPAD, 1), f32),
            jax.ShapeDtypeStruct((NPAD, 1), f32),
        ],
    )(o2, den2, b2.reshape(1, -1), w3, as3.reshape(-1, 1),
      ad3.reshape(-1, 1))


def _tc4_body(o3, den, b3, wlin2, blin2, wl4p, wr4, p4o, r4o):
    d = jnp.clip(jnp.sum(den[...], axis=0), 1e-16, None)
    o3s = jnp.sum(o3[...], axis=0)
    g3 = jnp.maximum(o3s / d[:, None] + b3[...], 0.0)
    h5 = jnp.maximum(
        jnp.dot(g3, wlin2[...], preferred_element_type=f32) + blin2[...], 0.0)
    p4o[...] = jnp.dot(h5, wl4p[...], preferred_element_type=f32)
    r4o[...] = jnp.dot(h5, wr4[...], preferred_element_type=f32)


def _tc4(o3, den3, b3, wlin2, blin2, wl4p, wr4):
    return pl.pallas_call(
        _tc4_body,
        grid=(40,),
        in_specs=[
            pl.BlockSpec((NC, MBLK, 128), lambda i: (0, i, 0)),
            pl.BlockSpec((NW, MBLK), lambda i: (0, i)),
            pl.BlockSpec((1, 128), lambda i: (0, 0)),
            pl.BlockSpec((128, 64), lambda i: (0, 0)),
            pl.BlockSpec((1, 64), lambda i: (0, 0)),
            pl.BlockSpec((64, 128), lambda i: (0, 0)),
            pl.BlockSpec((64, 4), lambda i: (0, 0)),
        ],
        out_specs=[
            pl.BlockSpec((MBLK, 128), lambda i: (i, 0)),
            pl.BlockSpec((MBLK, 4), lambda i: (i, 0)),
        ],
        out_shape=[
            jax.ShapeDtypeStruct((NPAD, 128), f32),
            jax.ShapeDtypeStruct((NPAD, 4), f32),
        ],
    )(o3, den3, b3.reshape(1, -1), wlin2, blin2.reshape(1, -1), wl4p, wr4)


def _tc5_body(s4, cnt, r4, bl4, o_ref):
    s = jnp.sum(s4[...], axis=0)
    c = jnp.clip(jnp.sum(cnt[...], axis=0), 1.0, None)
    mean = s[:, 0:4] / c[:, None]
    z = jnp.maximum(mean + bl4[...] + r4[...], 0.0)
    mz = jnp.max(z, axis=1, keepdims=True)
    o_ref[...] = z - mz - jnp.log(
        jnp.sum(jnp.exp(z - mz), axis=1, keepdims=True))


def _tc5(s4, cnt4, r4, bl4):
    return pl.pallas_call(
        _tc5_body,
        grid=(40,),
        in_specs=[
            pl.BlockSpec((NC, MBLK, 128), lambda i: (0, i, 0)),
            pl.BlockSpec((NW, MBLK), lambda i: (0, i)),
            pl.BlockSpec((MBLK, 4), lambda i: (i, 0)),
            pl.BlockSpec((1, 4), lambda i: (0, 0)),
        ],
        out_specs=pl.BlockSpec((MBLK, 4), lambda i: (i, 0)),
        out_shape=jax.ShapeDtypeStruct((NPAD, 4), f32),
    )(s4, cnt4, r4, bl4.reshape(1, -1))


def _pad_edges(src, dst, ep):
    e = src.shape[0]
    return (jnp.pad(src, (0, ep - e)),
            jnp.pad(dst, (0, ep - e), constant_values=DUMMY))


def kernel(x, edge_index_IVI, edge_index_IBI, edge_index_ITI,
           edge_index_IOI, Wl1, bl1, Wr1, Wlin1, blin1, W2, as2, ad2, b2,
           W3, as3, ad3, b3, Wlin2, blin2, Wl4, bl4, Wr4):
    src1, dst1 = _pad_edges(edge_index_IVI[0], edge_index_IVI[1], E1P)
    loop = jnp.arange(N, dtype=i32)
    sg = jnp.concatenate([edge_index_IVI[0], edge_index_ITI[0],
                          edge_index_IBI[0], edge_index_IOI[0], loop])
    dg = jnp.concatenate([edge_index_IVI[1], edge_index_ITI[1],
                          edge_index_IBI[1], edge_index_IOI[1], loop])
    srcg, dstg = _pad_edges(sg, dg, EGP)
    src4, dst4 = _pad_edges(edge_index_IBI[0], edge_index_IBI[1], E4P)

    xw = _tc1(x, jnp.concatenate([Wl1, Wr1], axis=1))
    hl = xw[:, :1024]
    hr = xw[:, 1024:]

    s1 = _row_scatter(hl, src1, dst1, None, nchunk=8, fc=128)
    cnt1 = _seg_count(dst1)
    h2, a2s, a2d = _tc2(s1, cnt1, hr, bl1, Wlin1, blin1, W2, as2, ad2)

    ex2, den2 = _gat_edge(a2s[:, 0], a2d[:, 0], srcg, dstg)
    o2 = _row_scatter(h2, srcg, dstg, ex2, nchunk=2, fc=128)
    h3, a3s, a3d = _tc3(o2, den2, b2, W3, as3, ad3)

    ex3, den3 = _gat_edge(a3s[:, 0], a3d[:, 0], srcg, dstg)
    o3 = _row_scatter(h3, srcg, dstg, ex3, nchunk=1, fc=128)

    wl4p = jnp.pad(Wl4, ((0, 0), (0, 124)))
    p4, r4 = _tc4(o3, den3, b3, Wlin2, blin2, wl4p, Wr4)

    s4 = _row_scatter(p4, src4, dst4, None, nchunk=1, fc=128)
    cnt4 = _seg_count(dst4)
    out = _tc5(s4, cnt4, r4, bl4)
    return out[:N]


# speedup vs baseline: 1.1338x; 1.1338x over previous
import functools

import jax
import jax.numpy as jnp
from jax import lax
from jax.experimental import pallas as pl
from jax.experimental.pallas import tpu as pltpu
from jax.experimental.pallas import tpu_sc as plsc

f32 = jnp.float32
i32 = jnp.int32

N = 10000
NPAD = 10240
DUMMY = 10000
MBLK = 256
NC, NS, NW = 2, 16, 32
EB = 256
E1P = 106496
E4P = 106496
EGP = 417792
RPS = NPAD // NS


_SC_PARAMS = pltpu.CompilerParams(needs_layout_passes=False)


def _mesh():
    return plsc.VectorSubcoreMesh(core_axis_name="c", subcore_axis_name="s")


def _row_scatter(table, src, dst, w, nchunk, fc):
    ep = src.shape[0]
    weighted = w is not None
    esplit = NW if nchunk == 1 else NS
    et = ep // esplit
    nb = et // EB

    if nchunk == 1:
        out_t = jax.ShapeDtypeStruct((NC, NPAD, fc), f32)
    else:
        out_t = jax.ShapeDtypeStruct((NPAD, nchunk * fc), f32)

    scratch = [
        pltpu.VMEM((EB,), i32),
        pltpu.VMEM((EB,), i32),
        pltpu.VMEM((EB, fc), f32),
        pltpu.VMEM_SHARED((NPAD, fc), f32),
        pltpu.SemaphoreType.DMA,
    ]
    if weighted:
        scratch.append(pltpu.VMEM((EB,), f32))

    def body(table_h, src_h, dst_h, *rest):
        if weighted:
            (w_h, out_h, sidx, didx, rows, acc, sem, wv) = rest
        else:
            (out_h, sidx, didx, rows, acc, sem) = rest
        c = lax.axis_index("c")
        s = lax.axis_index("s")
        zero16 = jnp.zeros((16,), f32)

        tid = s * NC + c if esplit == NW else s

        for q in range(max(nchunk // NC, 1)):
            ck = c + NC * q if nchunk > 1 else 0
            f0 = ck * fc

            def zr_body(r, _):
                for qq in range(fc // 16):
                    rows[r, pl.ds(qq * 16, 16)] = zero16
                return 0

            lax.fori_loop(0, EB, zr_body, 0, unroll=False)
            off = 0
            while off < RPS:
                n_ = min(EB, RPS - off)
                pltpu.sync_copy(rows.at[pl.ds(0, n_)],
                                acc.at[pl.ds(s * RPS + off, n_)])
                off += n_
            plsc.subcore_barrier()

            def e_body(b, _):
                e0 = tid * et + b * EB
                pltpu.sync_copy(src_h.at[pl.ds(e0, EB)], sidx)
                pltpu.sync_copy(dst_h.at[pl.ds(e0, EB)], didx)
                if nchunk > 1:
                    pltpu.async_copy(
                        table_h.at[sidx, pl.ds(f0, fc)], rows, sem
                    ).wait()
                else:
                    pltpu.async_copy(table_h.at[sidx], rows, sem).wait()
                if weighted:
                    pltpu.sync_copy(w_h.at[pl.ds(e0, EB)], wv)

                    def m_body(j, _):
                        wvec = wv[pl.ds(j * 16, 16)]
                        for l in range(16):
                            ws = wvec[l]
                            i = j * 16 + l
                            for qq in range(fc // 16):
                                sl = pl.ds(qq * 16, 16)
                                rows[i, sl] = rows[i, sl] * ws
                        return 0

                    lax.fori_loop(0, EB // 16, m_body, 0, unroll=False)
                pltpu.sync_copy(rows, acc.at[didx], add=True)
                return 0

            lax.fori_loop(0, nb, e_body, 0, unroll=False)
            plsc.subcore_barrier()

            r0_ = s * RPS
            if nchunk == 1:
                pltpu.sync_copy(acc.at[pl.ds(r0_, RPS)],
                                out_h.at[c, pl.ds(r0_, RPS)])
            else:
                pltpu.sync_copy(acc.at[pl.ds(r0_, RPS)],
                                out_h.at[pl.ds(r0_, RPS), pl.ds(f0, fc)])
            plsc.subcore_barrier()

    run = pl.kernel(body, out_type=out_t, mesh=_mesh(),
                    scratch_types=scratch, compiler_params=_SC_PARAMS)
    if weighted:
        return run(table, src, dst, w)
    return run(table, src, dst)


def _gat_edge(a_s, a_d, src, dst):
    ep = src.shape[0]
    et = ep // NW
    nb = et // EB

    out_t = (jax.ShapeDtypeStruct((ep,), f32),
             jax.ShapeDtypeStruct((NW, NPAD), f32))
    scratch = [
        pltpu.VMEM((NPAD,), f32),
        pltpu.VMEM((NPAD,), f32),
        pltpu.VMEM((NPAD,), f32),
        pltpu.VMEM((EB,), i32),
        pltpu.VMEM((EB,), i32),
        pltpu.VMEM((EB,), f32),
    ]

    def body(as_h, ad_h, src_h, dst_h, ex_h, den_h,
             asv, adv, acc, sidx, didx, exb):
        c = lax.axis_index("c")
        s = lax.axis_index("s")
        wid = s * NC + c
        pltpu.sync_copy(as_h, asv)
        pltpu.sync_copy(ad_h, adv)
        zero16 = jnp.zeros((16,), f32)

        def z_body(i, _):
            acc[pl.ds(i * 16, 16)] = zero16
            return 0

        lax.fori_loop(0, NPAD // 16, z_body, 0, unroll=False)

        def mx_body(i, cur):
            va = asv[pl.ds(i * 16, 16)]
            vb = adv[pl.ds(i * 16, 16)]
            return (jnp.maximum(cur[0], va), jnp.maximum(cur[1], vb))

        ma, mb = lax.fori_loop(0, NPAD // 16, mx_body,
                               (jnp.zeros((16,), f32), jnp.zeros((16,), f32)),
                               unroll=False)
        m = jnp.max(ma) + jnp.max(mb)

        def e_body(b, _):
            e0 = wid * et + b * EB
            pltpu.sync_copy(src_h.at[pl.ds(e0, EB)], sidx)
            pltpu.sync_copy(dst_h.at[pl.ds(e0, EB)], didx)
            for j in range(EB // 16):
                sl = pl.ds(j * 16, 16)
                si = sidx[sl]
                di = didx[sl]
                av = plsc.load_gather(asv, [si])
                bv = plsc.load_gather(adv, [di])
                t = av + bv
                al = jnp.maximum(t, 0.2 * t)
                e_ = jnp.exp(al - m)
                exb[sl] = e_
                plsc.addupdate_scatter(acc, [di], e_)
            pltpu.sync_copy(exb, ex_h.at[pl.ds(e0, EB)])
            return 0

        lax.fori_loop(0, nb, e_body, 0, unroll=False)
        pltpu.sync_copy(acc, den_h.at[wid])

    run = pl.kernel(body, out_type=out_t, mesh=_mesh(),
                    scratch_types=scratch, compiler_params=_SC_PARAMS)
    return run(a_s, a_d, src, dst)


def _seg_count(dst):
    ep = dst.shape[0]
    et = ep // NW
    nb = et // EB

    out_t = jax.ShapeDtypeStruct((NW, NPAD), f32)
    scratch = [
        pltpu.VMEM((NPAD,), f32),
        pltpu.VMEM((EB,), i32),
    ]

    def body(dst_h, out_h, acc, didx):
        c = lax.axis_index("c")
        s = lax.axis_index("s")
        wid = s * NC + c
        zero16 = jnp.zeros((16,), f32)
        ones16 = jnp.ones((16,), f32)

        def z_body(i, _):
            acc[pl.ds(i * 16, 16)] = zero16
            return 0

        lax.fori_loop(0, NPAD // 16, z_body, 0, unroll=False)

        def e_body(b, _):
            e0 = wid * et + b * EB
            pltpu.sync_copy(dst_h.at[pl.ds(e0, EB)], didx)
            for j in range(EB // 16):
                di = didx[pl.ds(j * 16, 16)]
                plsc.addupdate_scatter(acc, [di], ones16)
            return 0

        lax.fori_loop(0, nb, e_body, 0, unroll=False)
        pltpu.sync_copy(acc, out_h.at[wid])

    run = pl.kernel(body, out_type=out_t, mesh=_mesh(),
                    scratch_types=scratch, compiler_params=_SC_PARAMS)
    return run(dst)


def _tc1_body(x_ref, w_ref, o_ref):
    o_ref[...] = jnp.dot(x_ref[...], w_ref[...], preferred_element_type=f32)


def _tc1(x, w):
    return pl.pallas_call(
        _tc1_body,
        grid=(2, 40),
        in_specs=[
            pl.BlockSpec((MBLK, 2000), lambda n, m: (m, 0)),
            pl.BlockSpec((2000, 1024), lambda n, m: (0, n)),
        ],
        out_specs=pl.BlockSpec((MBLK, 1024), lambda n, m: (m, n)),
        out_shape=jax.ShapeDtypeStruct((NPAD, 2048), f32),
    )(x, w)


def _tc2_body(s1, cnt, hr, bl1, wlin1, blin1, w2, as2, ad2,
              h2o, aso, ado):
    c = jnp.clip(jnp.sum(cnt[...], axis=0), 1.0, None)
    h1 = jnp.maximum(s1[...] / c[:, None] + bl1[...] + hr[...], 0.0)
    h2 = jnp.maximum(
        jnp.dot(h1, wlin1[...], preferred_element_type=f32) + blin1[...], 0.0)
    hh = jnp.dot(h2, w2[...], preferred_element_type=f32)
    i = pl.program_id(0)
    rows = i * MBLK + lax.broadcasted_iota(i32, (MBLK, 1), 0)
    mask = rows < N
    hh = jnp.where(mask, hh, 0.0)
    h2o[...] = hh
    aso[...] = jnp.dot(hh, as2[...], preferred_element_type=f32)
    ado[...] = jnp.dot(hh, ad2[...], preferred_element_type=f32)


def _tc2(s1, cnt, hr, bl1, wlin1, blin1, w2, as2, ad2):
    return pl.pallas_call(
        _tc2_body,
        grid=(40,),
        in_specs=[
            pl.BlockSpec((MBLK, 1024), lambda i: (i, 0)),
            pl.BlockSpec((NW, MBLK), lambda i: (0, i)),
            pl.BlockSpec((MBLK, 1024), lambda i: (i, 0)),
            pl.BlockSpec((1, 1024), lambda i: (0, 0)),
            pl.BlockSpec((1024, 512), lambda i: (0, 0)),
            pl.BlockSpec((1, 512), lambda i: (0, 0)),
            pl.BlockSpec((512, 256), lambda i: (0, 0)),
            pl.BlockSpec((256, 1), lambda i: (0, 0)),
            pl.BlockSpec((256, 1), lambda i: (0, 0)),
        ],
        out_specs=[
            pl.BlockSpec((MBLK, 256), lambda i: (i, 0)),
            pl.BlockSpec((MBLK, 1), lambda i: (i, 0)),
            pl.BlockSpec((MBLK, 1), lambda i: (i, 0)),
        ],
        out_shape=[
            jax.ShapeDtypeStruct((NPAD, 256), f32),
            jax.ShapeDtypeStruct((NPAD, 1), f32),
            jax.ShapeDtypeStruct((NPAD, 1), f32),
        ],
    )(s1, cnt, hr, bl1.reshape(1, -1), wlin1, blin1.reshape(1, -1), w2,
      as2.reshape(-1, 1), ad2.reshape(-1, 1))


def _tc3_body(o2, den, b2, w3, as3, ad3, h3o, aso, ado):
    d = jnp.clip(jnp.sum(den[...], axis=0), 1e-16, None)
    g2 = jnp.maximum(o2[...] / d[:, None] + b2[...], 0.0)
    h3 = jnp.dot(g2, w3[...], preferred_element_type=f32)
    i = pl.program_id(0)
    rows = i * MBLK + lax.broadcasted_iota(i32, (MBLK, 1), 0)
    mask = rows < N
    h3 = jnp.where(mask, h3, 0.0)
    h3o[...] = h3
    aso[...] = jnp.dot(h3, as3[...], preferred_element_type=f32)
    ado[...] = jnp.dot(h3, ad3[...], preferred_element_type=f32)


def _tc3(o2, den2, b2, w3, as3, ad3):
    return pl.pallas_call(
        _tc3_body,
        grid=(40,),
        in_specs=[
            pl.BlockSpec((MBLK, 256), lambda i: (i, 0)),
            pl.BlockSpec((NW, MBLK), lambda i: (0, i)),
            pl.BlockSpec((1, 256), lambda i: (0, 0)),
            pl.BlockSpec((256, 128), lambda i: (0, 0)),
            pl.BlockSpec((128, 1), lambda i: (0, 0)),
            pl.BlockSpec((128, 1), lambda i: (0, 0)),
        ],
        out_specs=[
            pl.BlockSpec((MBLK, 128), lambda i: (i, 0)),
            pl.BlockSpec((MBLK, 1), lambda i: (i, 0)),
            pl.BlockSpec((MBLK, 1), lambda i: (i, 0)),
        ],
        out_shape=[
            jax.ShapeDtypeStruct((NPAD, 128), f32),
            jax.ShapeDtypeStruct((NPAD, 1), f32),
            jax.ShapeDtypeStruct((NPAD, 1), f32),
        ],
    )(o2, den2, b2.reshape(1, -1), w3, as3.reshape(-1, 1),
      ad3.reshape(-1, 1))


def _tc4_body(o3, den, b3, wlin2, blin2, wl4p, wr4, p4o, r4o):
    d = jnp.clip(jnp.sum(den[...], axis=0), 1e-16, None)
    o3s = jnp.sum(o3[...], axis=0)
    g3 = jnp.maximum(o3s / d[:, None] + b3[...], 0.0)
    h5 = jnp.maximum(
        jnp.dot(g3, wlin2[...], preferred_element_type=f32) + blin2[...], 0.0)
    p4o[...] = jnp.dot(h5, wl4p[...], preferred_element_type=f32)
    r4o[...] = jnp.dot(h5, wr4[...], preferred_element_type=f32)


def _tc4(o3, den3, b3, wlin2, blin2, wl4p, wr4):
    return pl.pallas_call(
        _tc4_body,
        grid=(40,),
        in_specs=[
            pl.BlockSpec((NC, MBLK, 128), lambda i: (0, i, 0)),
            pl.BlockSpec((NW, MBLK), lambda i: (0, i)),
            pl.BlockSpec((1, 128), lambda i: (0, 0)),
            pl.BlockSpec((128, 64), lambda i: (0, 0)),
            pl.BlockSpec((1, 64), lambda i: (0, 0)),
            pl.BlockSpec((64, 128), lambda i: (0, 0)),
            pl.BlockSpec((64, 4), lambda i: (0, 0)),
        ],
        out_specs=[
            pl.BlockSpec((MBLK, 128), lambda i: (i, 0)),
            pl.BlockSpec((MBLK, 4), lambda i: (i, 0)),
        ],
        out_shape=[
            jax.ShapeDtypeStruct((NPAD, 128), f32),
            jax.ShapeDtypeStruct((NPAD, 4), f32),
        ],
    )(o3, den3, b3.reshape(1, -1), wlin2, blin2.reshape(1, -1), wl4p, wr4)


def _tc5_body(s4, cnt, r4, bl4, o_ref):
    s = jnp.sum(s4[...], axis=0)
    c = jnp.clip(jnp.sum(cnt[...], axis=0), 1.0, None)
    mean = s[:, 0:4] / c[:, None]
    z = jnp.maximum(mean + bl4[...] + r4[...], 0.0)
    mz = jnp.max(z, axis=1, keepdims=True)
    o_ref[...] = z - mz - jnp.log(
        jnp.sum(jnp.exp(z - mz), axis=1, keepdims=True))


def _tc5(s4, cnt4, r4, bl4):
    return pl.pallas_call(
        _tc5_body,
        grid=(40,),
        in_specs=[
            pl.BlockSpec((NC, MBLK, 128), lambda i: (0, i, 0)),
            pl.BlockSpec((NW, MBLK), lambda i: (0, i)),
            pl.BlockSpec((MBLK, 4), lambda i: (i, 0)),
            pl.BlockSpec((1, 4), lambda i: (0, 0)),
        ],
        out_specs=pl.BlockSpec((MBLK, 4), lambda i: (i, 0)),
        out_shape=jax.ShapeDtypeStruct((NPAD, 4), f32),
    )(s4, cnt4, r4, bl4.reshape(1, -1))


def _pad_edges(src, dst, ep):
    e = src.shape[0]
    return (jnp.pad(src, (0, ep - e)),
            jnp.pad(dst, (0, ep - e), constant_values=DUMMY))


def kernel(x, edge_index_IVI, edge_index_IBI, edge_index_ITI,
           edge_index_IOI, Wl1, bl1, Wr1, Wlin1, blin1, W2, as2, ad2, b2,
           W3, as3, ad3, b3, Wlin2, blin2, Wl4, bl4, Wr4):
    src1, dst1 = _pad_edges(edge_index_IVI[0], edge_index_IVI[1], E1P)
    loop = jnp.arange(N, dtype=i32)
    sg = jnp.concatenate([edge_index_IVI[0], edge_index_ITI[0],
                          edge_index_IBI[0], edge_index_IOI[0], loop])
    dg = jnp.concatenate([edge_index_IVI[1], edge_index_ITI[1],
                          edge_index_IBI[1], edge_index_IOI[1], loop])
    srcg, dstg = _pad_edges(sg, dg, EGP)
    src4, dst4 = _pad_edges(edge_index_IBI[0], edge_index_IBI[1], E4P)

    xw = _tc1(x.astype(jnp.bfloat16),
              jnp.concatenate([Wl1, Wr1], axis=1).astype(jnp.bfloat16))
    hl = xw[:, :1024]
    hr = xw[:, 1024:]

    s1 = _row_scatter(hl, src1, dst1, None, nchunk=8, fc=128)
    cnt1 = _seg_count(dst1)
    h2, a2s, a2d = _tc2(s1, cnt1, hr, bl1, Wlin1, blin1, W2, as2, ad2)

    ex2, den2 = _gat_edge(a2s[:, 0], a2d[:, 0], srcg, dstg)
    o2 = _row_scatter(h2, srcg, dstg, ex2, nchunk=2, fc=128)
    h3, a3s, a3d = _tc3(o2, den2, b2, W3, as3, ad3)

    ex3, den3 = _gat_edge(a3s[:, 0], a3d[:, 0], srcg, dstg)
    o3 = _row_scatter(h3, srcg, dstg, ex3, nchunk=1, fc=128)

    wl4p = jnp.pad(Wl4, ((0, 0), (0, 124)))
    p4, r4 = _tc4(o3, den3, b3, Wlin2, blin2, wl4p, Wr4)

    s4 = _row_scatter(p4, src4, dst4, None, nchunk=1, fc=128)
    cnt4 = _seg_count(dst4)
    out = _tc5(s4, cnt4, r4, bl4)
    return out[:N]
